# Initial kernel scaffold; baseline (speedup 1.0000x reference)
import jax
import jax.numpy as jnp
from jax.experimental import pallas as pl


def kernel(seq, mods, aa_table, mod_table, pe):
    raise NotImplementedError("write your pallas kernel here")


# speedup vs baseline: 1.4849x; 1.4849x over previous
import functools
import math

import jax
import jax.numpy as jnp
from jax import lax
from jax.experimental import pallas as pl
from jax.experimental.pallas import tpu as pltpu
from jax.experimental.pallas import tpu_sc as plsc

D = 128
S = 50
B = 16384
NAA = 25
NMOD = 16
NF = NAA * NMOD
SCALE = math.sqrt(float(NAA))
TOK = S * B
L = 16
C = 128


def _make_sc_kernel():
    info = plsc.get_sparse_core_info()
    nc, ns = info.num_cores, info.num_subcores
    nw = nc * ns
    tpw = TOK // nw
    nchunk = tpw // C
    assert tpw * nw == TOK and nchunk * C == tpw and B % C == 0

    mesh = plsc.VectorSubcoreMesh(core_axis_name="c", subcore_axis_name="s")

    @functools.partial(
        pl.kernel,
        out_type=jax.ShapeDtypeStruct((TOK * D,), jnp.float32),
        mesh=mesh,
        compiler_params=pltpu.CompilerParams(needs_layout_passes=False),
        scratch_types=[
            pltpu.VMEM((NAA * D,), jnp.float32),
            pltpu.VMEM((NMOD * D,), jnp.float32),
            pltpu.VMEM((NF * D,), jnp.float32),
            pltpu.VMEM((S * D,), jnp.float32),
            pltpu.VMEM((C,), jnp.int32),
            pltpu.VMEM((C,), jnp.int32),
            pltpu.VMEM((C * D,), jnp.float32),
        ],
    )
    def sc_kernel(seq_hbm, mods_hbm, aa_hbm, mod_hbm, pe_hbm, out_hbm,
                  aa_v, mod_v, fused_v, pe_v, seq_v, mods_v, out_v):
        pltpu.sync_copy(aa_hbm, aa_v)
        pltpu.sync_copy(mod_hbm, mod_v)
        pltpu.sync_copy(pe_hbm, pe_v)
        for k in range(NAA * D // L):
            aa_v[pl.ds(k * L, L)] = aa_v[pl.ds(k * L, L)] * SCALE
        for k in range(NMOD * D // L):
            mod_v[pl.ds(k * L, L)] = mod_v[pl.ds(k * L, L)] * SCALE

        wid = lax.axis_index("s") * nc + lax.axis_index("c")
        t0w = wid * tpw

        def rebuild(pos):
            pe_regs = [pe_v[pl.ds(pos * D + j * L, L)] for j in range(D // L)]

            def build(a, carry):
                base = a * (NMOD * D)
                for j in range(D // L):
                    aape_j = aa_v[pl.ds(a * D + j * L, L)] + pe_regs[j]
                    for m in range(NMOD):
                        fused_v[pl.ds(base + m * D + j * L, L)] = (
                            aape_j + mod_v[pl.ds(m * D + j * L, L)]
                        )
                return carry
            lax.fori_loop(0, NAA, build, 0)

        lane = lax.iota(jnp.int32, L)
        st_base = lane * D

        def chunk_body(g, prev_pos):
            t0 = t0w + g * C
            pos = t0 // B

            @pl.when(pos != prev_pos)
            def _():
                rebuild(pos)

            pltpu.sync_copy(seq_hbm.at[pl.ds(t0, C)], seq_v)
            pltpu.sync_copy(mods_hbm.at[pl.ds(t0, C)], mods_v)

            for g2 in range(C // L):
                sv = seq_v[pl.ds(g2 * L, L)]
                mv = mods_v[pl.ds(g2 * L, L)]
                off = (sv * NMOD + mv) * D
                stb = st_base + (g2 * L * D)

                def col(c, c2):
                    v = plsc.load_gather(fused_v, (off + c,))
                    plsc.store_scatter(out_v, (stb + c,), v)
                    return c2
                lax.fori_loop(0, D, col, 0, unroll=4)

            pltpu.sync_copy(out_v, out_hbm.at[pl.ds(t0 * D, C * D)])
            return pos
        lax.fori_loop(0, nchunk, chunk_body, jnp.int32(-1))

    return sc_kernel


_SC_KERNEL = _make_sc_kernel()


def kernel(seq, mods, aa_table, mod_table, pe):
    seq_t = seq.T.reshape(-1).astype(jnp.int32)
    mods_t = mods.T.reshape(-1).astype(jnp.int32)
    out = _SC_KERNEL(seq_t, mods_t,
                     aa_table.reshape(-1), mod_table.reshape(-1),
                     pe.reshape(-1))
    return out.reshape(S, B, D)


# speedup vs baseline: 2.2336x; 1.5042x over previous
import functools
import math

import jax
import jax.numpy as jnp
from jax import lax
from jax.experimental import pallas as pl
from jax.experimental.pallas import tpu as pltpu
from jax.experimental.pallas import tpu_sc as plsc

D = 128
S = 50
B = 16384
NAA = 25
NMOD = 16
NF = NAA * NMOD
SCALE = math.sqrt(float(NAA))
TOK = S * B
L = 16
C = 256
CD = C * D


def _make_sc_kernel():
    info = plsc.get_sparse_core_info()
    nc, ns = info.num_cores, info.num_subcores
    nw = nc * ns
    tpw = TOK // nw
    nchunk = tpw // C
    npair = nchunk // 2
    assert tpw * nw == TOK and npair * 2 * C == tpw
    assert B % (2 * C) == 0

    mesh = plsc.VectorSubcoreMesh(core_axis_name="c", subcore_axis_name="s")

    @functools.partial(
        pl.kernel,
        out_type=jax.ShapeDtypeStruct((TOK * D,), jnp.float32),
        mesh=mesh,
        compiler_params=pltpu.CompilerParams(needs_layout_passes=False),
        scratch_types=[
            pltpu.VMEM((NAA * D,), jnp.float32),
            pltpu.VMEM((NMOD * D,), jnp.float32),
            pltpu.VMEM((NF * D,), jnp.float32),
            pltpu.VMEM((S * D,), jnp.float32),
            pltpu.VMEM((C,), jnp.int32),
            pltpu.VMEM((C,), jnp.int32),
            pltpu.VMEM((C,), jnp.int32),
            pltpu.VMEM((C,), jnp.int32),
            pltpu.VMEM((CD,), jnp.float32),
            pltpu.VMEM((CD,), jnp.float32),
            pltpu.SemaphoreType.DMA,
            pltpu.SemaphoreType.DMA,
            pltpu.SemaphoreType.DMA,
            pltpu.SemaphoreType.DMA,
        ],
    )
    def sc_kernel(seq_hbm, mods_hbm, aa_hbm, mod_hbm, pe_hbm, out_hbm,
                  aa_v, mod_v, fused_v, pe_v, s0v, s1v, m0v, m1v,
                  o0v, o1v, si0, si1, so0, so1):
        pltpu.sync_copy(aa_hbm, aa_v)
        pltpu.sync_copy(mod_hbm, mod_v)
        pltpu.sync_copy(pe_hbm, pe_v)
        for k in range(NAA * D // L):
            aa_v[pl.ds(k * L, L)] = aa_v[pl.ds(k * L, L)] * SCALE
        for k in range(NMOD * D // L):
            mod_v[pl.ds(k * L, L)] = mod_v[pl.ds(k * L, L)] * SCALE

        wid = lax.axis_index("s") * nc + lax.axis_index("c")
        t0w = wid * tpw

        def rebuild(pos):
            pe_regs = [pe_v[pl.ds(pos * D + j * L, L)] for j in range(D // L)]

            def build(a, carry):
                base = a * (NMOD * D)
                for j in range(D // L):
                    aape_j = aa_v[pl.ds(a * D + j * L, L)] + pe_regs[j]
                    for m in range(NMOD):
                        fused_v[pl.ds(base + m * D + j * L, L)] = (
                            aape_j + mod_v[pl.ds(m * D + j * L, L)]
                        )
                return carry
            lax.fori_loop(0, NAA, build, 0)

        lane = lax.iota(jnp.int32, L)
        st_base = lane * D

        def compute_chunk(sv_ref, mv_ref, ov_ref):
            for g2 in range(C // L):
                sv = sv_ref[pl.ds(g2 * L, L)]
                mv = mv_ref[pl.ds(g2 * L, L)]
                off = (sv * NMOD + mv) * D
                stb = st_base + (g2 * L * D)

                def col(c, c2):
                    c8 = c * 8
                    vs = [plsc.load_gather(fused_v, (off + (c8 + k),))
                          for k in range(8)]
                    for k in range(8):
                        plsc.store_scatter(ov_ref, (stb + (c8 + k),), vs[k])
                    return c2
                lax.fori_loop(0, D // 8, col, 0)

        pltpu.async_copy(seq_hbm.at[pl.ds(t0w, C)], s0v, si0)
        pltpu.async_copy(mods_hbm.at[pl.ds(t0w, C)], m0v, si0)

        def pair(h, prev_pos):
            t0 = t0w + h * (2 * C)
            t1 = t0 + C
            pos = t0 // B

            @pl.when(pos != prev_pos)
            def _():
                rebuild(pos)

            pltpu.make_async_copy(seq_hbm.at[pl.ds(t0, C)], s0v, si0).wait()
            pltpu.make_async_copy(mods_hbm.at[pl.ds(t0, C)], m0v, si0).wait()
            pltpu.async_copy(seq_hbm.at[pl.ds(t1, C)], s1v, si1)
            pltpu.async_copy(mods_hbm.at[pl.ds(t1, C)], m1v, si1)

            @pl.when(h > 0)
            def _():
                pltpu.make_async_copy(
                    o0v, out_hbm.at[pl.ds(t0 * D, CD)], so0).wait()

            compute_chunk(s0v, m0v, o0v)
            pltpu.async_copy(o0v, out_hbm.at[pl.ds(t0 * D, CD)], so0)

            @pl.when(h + 1 < npair)
            def _():
                tn = t0 + 2 * C
                pltpu.async_copy(seq_hbm.at[pl.ds(tn, C)], s0v, si0)
                pltpu.async_copy(mods_hbm.at[pl.ds(tn, C)], m0v, si0)

            pltpu.make_async_copy(seq_hbm.at[pl.ds(t1, C)], s1v, si1).wait()
            pltpu.make_async_copy(mods_hbm.at[pl.ds(t1, C)], m1v, si1).wait()

            @pl.when(h > 0)
            def _():
                pltpu.make_async_copy(
                    o1v, out_hbm.at[pl.ds(t1 * D, CD)], so1).wait()

            compute_chunk(s1v, m1v, o1v)
            pltpu.async_copy(o1v, out_hbm.at[pl.ds(t1 * D, CD)], so1)
            return pos

        lax.fori_loop(0, npair, pair, jnp.int32(-1))

        pltpu.make_async_copy(o0v, out_hbm.at[pl.ds(t0w * D, CD)], so0).wait()
        pltpu.make_async_copy(o1v, out_hbm.at[pl.ds(t0w * D, CD)], so1).wait()

    return sc_kernel


_SC_KERNEL = _make_sc_kernel()


def kernel(seq, mods, aa_table, mod_table, pe):
    seq_t = seq.T.reshape(-1).astype(jnp.int32)
    mods_t = mods.T.reshape(-1).astype(jnp.int32)
    out = _SC_KERNEL(seq_t, mods_t,
                     aa_table.reshape(-1), mod_table.reshape(-1),
                     pe.reshape(-1))
    return out.reshape(S, B, D)


# speedup vs baseline: 22.4570x; 10.0540x over previous
import functools
import math

import jax
import jax.numpy as jnp
from jax import lax
from jax.experimental import pallas as pl
from jax.experimental.pallas import tpu as pltpu
from jax.experimental.pallas import tpu_sc as plsc

D = 128
S = 50
B = 16384
NAA = 25
NMOD = 16
NF = NAA * NMOD
SCALE = math.sqrt(float(NAA))
TOK = S * B
L = 16
C = 256
CD = C * D


def _make_sc_kernel():
    info = plsc.get_sparse_core_info()
    nc, ns = info.num_cores, info.num_subcores
    nw = nc * ns
    tpw = TOK // nw
    nchunk = tpw // C
    npair = nchunk // 2
    assert tpw * nw == TOK and npair * 2 * C == tpw
    assert B % (2 * C) == 0

    mesh = plsc.VectorSubcoreMesh(core_axis_name="c", subcore_axis_name="s")

    @functools.partial(
        pl.kernel,
        out_type=jax.ShapeDtypeStruct((TOK * D,), jnp.float32),
        mesh=mesh,
        compiler_params=pltpu.CompilerParams(needs_layout_passes=False),
        scratch_types=[
            pltpu.VMEM((NAA * D,), jnp.float32),
            pltpu.VMEM((NMOD * D,), jnp.float32),
            pltpu.VMEM((NF * D,), jnp.float32),
            pltpu.VMEM((S * D,), jnp.float32),
            pltpu.VMEM((C,), jnp.int32),
            pltpu.VMEM((C,), jnp.int32),
            pltpu.VMEM((C,), jnp.int32),
            pltpu.VMEM((C,), jnp.int32),
            pltpu.VMEM((CD,), jnp.float32),
            pltpu.VMEM((CD,), jnp.float32),
            pltpu.SemaphoreType.DMA,
            pltpu.SemaphoreType.DMA,
            pltpu.SemaphoreType.DMA,
            pltpu.SemaphoreType.DMA,
        ],
    )
    def sc_kernel(seq_hbm, mods_hbm, aa_hbm, mod_hbm, pe_hbm, out_hbm,
                  aa_v, mod_v, fused_v, pe_v, s0v, s1v, m0v, m1v,
                  o0v, o1v, si0, si1, so0, so1):
        pltpu.sync_copy(aa_hbm, aa_v)
        pltpu.sync_copy(mod_hbm, mod_v)
        pltpu.sync_copy(pe_hbm, pe_v)
        for k in range(NAA * D // L):
            aa_v[pl.ds(k * L, L)] = aa_v[pl.ds(k * L, L)] * SCALE
        for k in range(NMOD * D // L):
            mod_v[pl.ds(k * L, L)] = mod_v[pl.ds(k * L, L)] * SCALE

        wid = lax.axis_index("s") * nc + lax.axis_index("c")
        t0w = wid * tpw

        def rebuild(pos):
            pe_regs = [pe_v[pl.ds(pos * D + j * L, L)] for j in range(D // L)]

            def build(a, carry):
                base = a * (NMOD * D)
                for j in range(D // L):
                    aape_j = aa_v[pl.ds(a * D + j * L, L)] + pe_regs[j]
                    for m in range(NMOD):
                        fused_v[pl.ds(base + m * D + j * L, L)] = (
                            aape_j + mod_v[pl.ds(m * D + j * L, L)]
                        )
                return carry
            lax.fori_loop(0, NAA, build, 0)

        def compute_chunk(sv_ref, mv_ref, ov_ref):
            def group(g2, c2):
                sv = sv_ref[pl.ds(g2 * L, L)]
                mv = mv_ref[pl.ds(g2 * L, L)]
                off = (sv * NMOD + mv) * D
                gbase = g2 * (L * D)
                prev_rows, prev_base = None, 0
                for k in range(L):
                    ok = off[k]
                    rows = [fused_v[pl.ds(ok + j * L, L)]
                            for j in range(D // L)]
                    if prev_rows is not None:
                        for j in range(D // L):
                            ov_ref[pl.ds(prev_base + j * L, L)] = prev_rows[j]
                    prev_rows, prev_base = rows, gbase + k * D
                for j in range(D // L):
                    ov_ref[pl.ds(prev_base + j * L, L)] = prev_rows[j]
                return c2
            lax.fori_loop(0, C // L, group, 0)

        pltpu.async_copy(seq_hbm.at[pl.ds(t0w, C)], s0v, si0)
        pltpu.async_copy(mods_hbm.at[pl.ds(t0w, C)], m0v, si0)

        def pair(h, prev_pos):
            t0 = t0w + h * (2 * C)
            t1 = t0 + C
            pos = t0 // B

            @pl.when(pos != prev_pos)
            def _():
                rebuild(pos)

            pltpu.make_async_copy(seq_hbm.at[pl.ds(t0, C)], s0v, si0).wait()
            pltpu.make_async_copy(mods_hbm.at[pl.ds(t0, C)], m0v, si0).wait()
            pltpu.async_copy(seq_hbm.at[pl.ds(t1, C)], s1v, si1)
            pltpu.async_copy(mods_hbm.at[pl.ds(t1, C)], m1v, si1)

            @pl.when(h > 0)
            def _():
                pltpu.make_async_copy(
                    o0v, out_hbm.at[pl.ds(t0 * D, CD)], so0).wait()

            compute_chunk(s0v, m0v, o0v)
            pltpu.async_copy(o0v, out_hbm.at[pl.ds(t0 * D, CD)], so0)

            @pl.when(h + 1 < npair)
            def _():
                tn = t0 + 2 * C
                pltpu.async_copy(seq_hbm.at[pl.ds(tn, C)], s0v, si0)
                pltpu.async_copy(mods_hbm.at[pl.ds(tn, C)], m0v, si0)

            pltpu.make_async_copy(seq_hbm.at[pl.ds(t1, C)], s1v, si1).wait()
            pltpu.make_async_copy(mods_hbm.at[pl.ds(t1, C)], m1v, si1).wait()

            @pl.when(h > 0)
            def _():
                pltpu.make_async_copy(
                    o1v, out_hbm.at[pl.ds(t1 * D, CD)], so1).wait()

            compute_chunk(s1v, m1v, o1v)
            pltpu.async_copy(o1v, out_hbm.at[pl.ds(t1 * D, CD)], so1)
            return pos

        lax.fori_loop(0, npair, pair, jnp.int32(-1))

        pltpu.make_async_copy(o0v, out_hbm.at[pl.ds(t0w * D, CD)], so0).wait()
        pltpu.make_async_copy(o1v, out_hbm.at[pl.ds(t0w * D, CD)], so1).wait()

    return sc_kernel


_SC_KERNEL = _make_sc_kernel()


def kernel(seq, mods, aa_table, mod_table, pe):
    seq_t = seq.T.reshape(-1).astype(jnp.int32)
    mods_t = mods.T.reshape(-1).astype(jnp.int32)
    out = _SC_KERNEL(seq_t, mods_t,
                     aa_table.reshape(-1), mod_table.reshape(-1),
                     pe.reshape(-1))
    return out.reshape(S, B, D)
